# baseline (speedup 1.0000x reference)
import functools

import jax
import jax.numpy as jnp
from jax import lax
from jax.experimental import pallas as pl
from jax.experimental.pallas import tpu as pltpu
from jax.experimental.pallas import tpu_sc as plsc

N = 131072
D = 32
K = 512


R = 1024
NB = N // R


NC = 2
NS = 16
NW = NC * NS
RPW = N // NW
RPC = 512
NCH = RPW // RPC
GPC = RPC // 16
CW = RPC * D


def _dist_argmin_kernel(x_ref, c_ref, ids_ref):
    x = x_ref[...]
    c = c_ref[...]
    cnorm = jnp.sum(c * c, axis=0, keepdims=True)
    xc2 = jnp.dot(x, -2.0 * c, preferred_element_type=jnp.float32)
    dist = xc2 + cnorm
    ids_ref[...] = jnp.argmin(dist, axis=1).astype(jnp.int32)


_dist_argmin = pl.pallas_call(
    _dist_argmin_kernel,
    grid=(NB,),
    in_specs=[
        pl.BlockSpec((R, D), lambda i: (i, 0)),
        pl.BlockSpec((D, K), lambda i: (0, 0)),
    ],
    out_specs=pl.BlockSpec((R,), lambda i: (i,)),
    out_shape=jax.ShapeDtypeStruct((N,), jnp.int32),
)


_sc_mesh = plsc.VectorSubcoreMesh(core_axis_name="c", subcore_axis_name="s")


@functools.partial(
    pl.kernel,
    mesh=_sc_mesh,
    out_type=jax.ShapeDtypeStruct((N * D,), jnp.float32),
    scratch_types=[
        pltpu.VMEM((D, K), jnp.float32),
        pltpu.VMEM((RPW,), jnp.int32),
        pltpu.VMEM((CW,), jnp.float32),
    ],
    compiler_params=pltpu.CompilerParams(needs_layout_passes=False),
)
def _gather_codewords(table_hbm, idx_hbm, out_hbm, table_v, idx_v, out_c):
    wid = lax.axis_index("s") * NC + lax.axis_index("c")
    pltpu.sync_copy(table_hbm, table_v)
    pltpu.sync_copy(idx_hbm.at[pl.ds(wid * RPW, RPW)], idx_v)
    pos0 = lax.iota(jnp.int32, 16) * D

    def chunk_body(k, carry):
        @plsc.parallel_loop(0, GPC, unroll=2)
        def group_body(g):
            ids16 = idx_v[pl.ds(k * RPC + g * 16, 16)]
            pos = g * (16 * D) + pos0
            for col in range(D):
                col_vec = jnp.full((16,), col, jnp.int32)
                v = plsc.load_gather(table_v, [col_vec, ids16])
                plsc.store_scatter(out_c, [pos + col], v)

        pltpu.sync_copy(out_c, out_hbm.at[pl.ds((wid * NCH + k) * CW, CW)])
        return carry

    lax.fori_loop(0, NCH, chunk_body, 0)


def kernel(x, C):
    ids = _dist_argmin(x, C)
    out = _gather_codewords(C, ids)
    return out.reshape(N, D)


# speedup vs baseline: 1.0588x; 1.0588x over previous
import functools

import jax
import jax.numpy as jnp
from jax import lax
from jax.experimental import pallas as pl
from jax.experimental.pallas import tpu as pltpu
from jax.experimental.pallas import tpu_sc as plsc

N = 131072
D = 32
K = 512


R = 1024
NB = N // R


NC = 2
NS = 16
NW = NC * NS
RPW = N // NW
RPC = 512
NCH = RPW // RPC
GPC = RPC // 16
CW = RPC * D


def _dist_argmin_kernel(x_ref, c_ref, ids_ref):
    x = x_ref[...]
    c = c_ref[...]
    cnorm = jnp.sum(c * c, axis=0, keepdims=True)
    xc2 = jnp.dot(x, -2.0 * c, preferred_element_type=jnp.float32)
    dist = xc2 + cnorm
    ids_ref[...] = jnp.argmin(dist, axis=1).astype(jnp.int32)


_dist_argmin = pl.pallas_call(
    _dist_argmin_kernel,
    grid=(NB,),
    in_specs=[
        pl.BlockSpec((R, D), lambda i: (i, 0)),
        pl.BlockSpec((D, K), lambda i: (0, 0)),
    ],
    out_specs=pl.BlockSpec((R,), lambda i: (i,)),
    out_shape=jax.ShapeDtypeStruct((N,), jnp.int32),
)


_sc_mesh = plsc.VectorSubcoreMesh(core_axis_name="c", subcore_axis_name="s")


@functools.partial(
    pl.kernel,
    mesh=_sc_mesh,
    out_type=jax.ShapeDtypeStruct((N, D), jnp.float32),
    scratch_types=[
        pltpu.VMEM((D, K), jnp.float32),
        pltpu.VMEM((RPW,), jnp.int32),
        pltpu.VMEM((RPC, D), jnp.float32),
    ],
    compiler_params=pltpu.CompilerParams(needs_layout_passes=False),
)
def _gather_codewords(table_hbm, idx_hbm, out_hbm, table_v, idx_v, out_c):
    wid = lax.axis_index("s") * NC + lax.axis_index("c")
    pltpu.sync_copy(table_hbm, table_v)
    pltpu.sync_copy(idx_hbm.at[pl.ds(wid * RPW, RPW)], idx_v)
    lane = lax.iota(jnp.int32, 16)

    def chunk_body(k, carry):
        @plsc.parallel_loop(0, GPC, unroll=2)
        def group_body(g):
            ids16 = idx_v[pl.ds(k * RPC + g * 16, 16)]
            rows16 = g * 16 + lane
            for col in range(D):
                col_vec = jnp.full((16,), col, jnp.int32)
                v = plsc.load_gather(table_v, [col_vec, ids16])
                plsc.store_scatter(out_c, [rows16, col_vec], v)

        pltpu.sync_copy(out_c, out_hbm.at[pl.ds((wid * NCH + k) * RPC, RPC)])
        return carry

    lax.fori_loop(0, NCH, chunk_body, 0)


def kernel(x, C):
    ids = _dist_argmin(x, C)
    return _gather_codewords(C, ids)


# speedup vs baseline: 1.1041x; 1.0427x over previous
import functools

import jax
import jax.numpy as jnp
from jax import lax
from jax.experimental import pallas as pl
from jax.experimental.pallas import tpu as pltpu
from jax.experimental.pallas import tpu_sc as plsc

N = 131072
D = 32
K = 512

P = 2
NP = N // P


R = 1024
NBP = NP // R


NC = 2
NS = 16
NW = NC * NS
RPW = NP // NW
RPC = 512
NCH = RPW // RPC
GPC = RPC // 16


def _dist_argmin_kernel(x_ref, c_ref, ids_ref):
    x = x_ref[...]
    c = c_ref[...]
    cnorm = jnp.sum(c * c, axis=0, keepdims=True)
    xc2 = jnp.dot(x, -2.0 * c, preferred_element_type=jnp.float32)
    dist = xc2 + cnorm
    ids_ref[...] = jnp.argmin(dist, axis=1).astype(jnp.int32)


def _make_dist_argmin(phase):
    return pl.pallas_call(
        _dist_argmin_kernel,
        grid=(NBP,),
        in_specs=[
            pl.BlockSpec((R, D), lambda i, o=phase * NBP: (i + o, 0)),
            pl.BlockSpec((D, K), lambda i: (0, 0)),
        ],
        out_specs=pl.BlockSpec((R,), lambda i: (i,)),
        out_shape=jax.ShapeDtypeStruct((NP,), jnp.int32),
    )


_sc_mesh = plsc.VectorSubcoreMesh(core_axis_name="c", subcore_axis_name="s")


def _make_gather(phase):
    row_base = phase * NP

    @functools.partial(
        pl.kernel,
        mesh=_sc_mesh,
        out_type=(),
        scratch_types=[
            pltpu.VMEM((D, K), jnp.float32),
            pltpu.VMEM((RPW,), jnp.int32),
            pltpu.VMEM((RPC, D), jnp.float32),
        ],
        compiler_params=pltpu.CompilerParams(needs_layout_passes=False),
    )
    def _gather_codewords(table_hbm, idx_hbm, out_hbm, table_v, idx_v, out_c):
        wid = lax.axis_index("s") * NC + lax.axis_index("c")
        pltpu.sync_copy(table_hbm, table_v)
        pltpu.sync_copy(idx_hbm.at[pl.ds(wid * RPW, RPW)], idx_v)
        lane = lax.iota(jnp.int32, 16)

        def chunk_body(k, carry):
            @plsc.parallel_loop(0, GPC, unroll=2)
            def group_body(g):
                ids16 = idx_v[pl.ds(k * RPC + g * 16, 16)]
                rows16 = g * 16 + lane
                for col in range(D):
                    col_vec = jnp.full((16,), col, jnp.int32)
                    v = plsc.load_gather(table_v, [col_vec, ids16])
                    plsc.store_scatter(out_c, [rows16, col_vec], v)

            row0 = row_base + (wid * NCH + k) * RPC
            pltpu.sync_copy(out_c, out_hbm.at[pl.ds(row0, RPC)])
            return carry

        lax.fori_loop(0, NCH, chunk_body, 0)

    return _gather_codewords


_tc_calls = [_make_dist_argmin(p) for p in range(P)]
_sc_calls = [_make_gather(p) for p in range(P)]


def kernel(x, C):
    out_ref = jax.new_ref(jnp.zeros((N, D), jnp.float32))
    for p in range(P):
        ids = _tc_calls[p](x, C)
        _sc_calls[p](C, ids, out_ref)
    return out_ref[...]


# speedup vs baseline: 1.1827x; 1.0712x over previous
import functools

import jax
import jax.numpy as jnp
from jax import lax
from jax.experimental import pallas as pl
from jax.experimental.pallas import tpu as pltpu
from jax.experimental.pallas import tpu_sc as plsc
from jax._src.pallas import mpmd as _pl_mpmd

N = 131072
D = 32
K = 512

P = 2
NP = N // P


R = 1024
NBP = NP // R


NC = 2
NS = 16
NW = NC * NS
RPW = NP // NW
RPC = 512
NCH = RPW // RPC
GPC = RPC // 16


def _dist_argmin_kernel(x_ref, c_ref, ids_ref):
    x = x_ref[...]
    c = c_ref[...]
    cnorm = jnp.sum(c * c, axis=0, keepdims=True)
    xc2 = jnp.dot(x, -2.0 * c, preferred_element_type=jnp.float32)
    dist = xc2 + cnorm
    ids_ref[...] = jnp.argmin(dist, axis=1).astype(jnp.int32)


def _make_dist_argmin(phase):
    return pl.pallas_call(
        _dist_argmin_kernel,
        grid=(NBP,),
        in_specs=[
            pl.BlockSpec((R, D), lambda i, o=phase * NBP: (i + o, 0)),
            pl.BlockSpec((D, K), lambda i: (0, 0)),
        ],
        out_specs=pl.BlockSpec((R,), lambda i: (i,)),
        out_shape=jax.ShapeDtypeStruct((NP,), jnp.int32),
    )


_sc_mesh = plsc.VectorSubcoreMesh(core_axis_name="c", subcore_axis_name="s")


def _make_gather(phase):
    row_base = phase * NP
    aliased = phase > 0

    def _gather_codewords(table_hbm, idx_hbm, *rest):
        if aliased:
            _, out_hbm, table_v, idx_v, out_c = rest
        else:
            out_hbm, table_v, idx_v, out_c = rest
        wid = lax.axis_index("s") * NC + lax.axis_index("c")
        pltpu.sync_copy(table_hbm, table_v)
        pltpu.sync_copy(idx_hbm.at[pl.ds(wid * RPW, RPW)], idx_v)
        lane = lax.iota(jnp.int32, 16)

        def chunk_body(k, carry):
            @plsc.parallel_loop(0, GPC, unroll=2)
            def group_body(g):
                ids16 = idx_v[pl.ds(k * RPC + g * 16, 16)]
                rows16 = g * 16 + lane
                for col in range(D):
                    col_vec = jnp.full((16,), col, jnp.int32)
                    v = plsc.load_gather(table_v, [col_vec, ids16])
                    plsc.store_scatter(out_c, [rows16, col_vec], v)

            row0 = row_base + (wid * NCH + k) * RPC
            pltpu.sync_copy(out_c, out_hbm.at[pl.ds(row0, RPC)])
            return carry

        lax.fori_loop(0, NCH, chunk_body, 0)

    return _pl_mpmd._mpmd_map(
        [(_sc_mesh, _gather_codewords)],
        jax.ShapeDtypeStruct((N, D), jnp.float32),
        input_output_aliases={2: 0} if aliased else {},
        scratch_types=[
            pltpu.VMEM((D, K), jnp.float32),
            pltpu.VMEM((RPW,), jnp.int32),
            pltpu.VMEM((RPC, D), jnp.float32),
        ],
        compiler_params=pltpu.CompilerParams(needs_layout_passes=False),
    )


_tc_calls = [_make_dist_argmin(p) for p in range(P)]
_sc_calls = [_make_gather(p) for p in range(P)]


def kernel(x, C):
    out = None
    for p in range(P):
        ids = _tc_calls[p](x, C)
        out = _sc_calls[p](C, ids) if p == 0 else _sc_calls[p](C, ids, out)
    return out


# speedup vs baseline: 1.2488x; 1.0559x over previous
import functools

import jax
import jax.numpy as jnp
from jax import lax
from jax.experimental import pallas as pl
from jax.experimental.pallas import tpu as pltpu
from jax.experimental.pallas import tpu_sc as plsc
from jax._src.pallas import mpmd as _pl_mpmd

N = 131072
D = 32
K = 512

P = 2
NP = N // P


R = 1024
NBP = NP // R


NC = 2
NS = 16
NW = NC * NS
RPW = NP // NW
RPC = 512
NCH = RPW // RPC
GPC = RPC // 16


def _dist_argmin_kernel(xt_ref, c_ref, ids_ref):
    xt = xt_ref[...]
    c = c_ref[...]
    cnorm = jnp.sum(c * c, axis=0, keepdims=True)
    xc2 = lax.dot_general(
        xt, -2.0 * c, (((0,), (0,)), ((), ())),
        preferred_element_type=jnp.float32,
    )
    dist = xc2 + cnorm
    ids_ref[...] = jnp.argmin(dist, axis=1).astype(jnp.int32)


def _make_dist_argmin(phase):
    return pl.pallas_call(
        _dist_argmin_kernel,
        grid=(NBP,),
        in_specs=[
            pl.BlockSpec((D, R), lambda i, o=phase * NBP: (0, i + o)),
            pl.BlockSpec((D, K), lambda i: (0, 0)),
        ],
        out_specs=pl.BlockSpec((R,), lambda i: (i,)),
        out_shape=jax.ShapeDtypeStruct((NP,), jnp.int32),
    )


_sc_mesh = plsc.VectorSubcoreMesh(core_axis_name="c", subcore_axis_name="s")


def _make_gather(phase):
    col_base = phase * NP
    aliased = phase > 0

    def _gather_codewords(table_hbm, idx_hbm, *rest):
        if aliased:
            _, out_hbm, table_v, idx_v, out_c = rest
        else:
            out_hbm, table_v, idx_v, out_c = rest
        wid = lax.axis_index("s") * NC + lax.axis_index("c")
        pltpu.sync_copy(table_hbm, table_v)
        pltpu.sync_copy(idx_hbm.at[pl.ds(wid * RPW, RPW)], idx_v)

        def chunk_body(k, carry):
            @plsc.parallel_loop(0, GPC, unroll=2)
            def group_body(g):
                ids16 = idx_v[pl.ds(k * RPC + g * 16, 16)]
                for col in range(D):
                    col_vec = jnp.full((16,), col, jnp.int32)
                    v = plsc.load_gather(table_v, [col_vec, ids16])
                    out_c[col, pl.ds(g * 16, 16)] = v

            col0 = col_base + (wid * NCH + k) * RPC
            pltpu.sync_copy(out_c, out_hbm.at[:, pl.ds(col0, RPC)])
            return carry

        lax.fori_loop(0, NCH, chunk_body, 0)

    return _pl_mpmd._mpmd_map(
        [(_sc_mesh, _gather_codewords)],
        jax.ShapeDtypeStruct((D, N), jnp.float32),
        input_output_aliases={2: 0} if aliased else {},
        scratch_types=[
            pltpu.VMEM((D, K), jnp.float32),
            pltpu.VMEM((RPW,), jnp.int32),
            pltpu.VMEM((D, RPC), jnp.float32),
        ],
        compiler_params=pltpu.CompilerParams(needs_layout_passes=False),
    )


_tc_calls = [_make_dist_argmin(p) for p in range(P)]
_sc_calls = [_make_gather(p) for p in range(P)]


def kernel(x, C):
    xt = x.T
    out_t = None
    for p in range(P):
        ids = _tc_calls[p](xt, C)
        out_t = _sc_calls[p](C, ids) if p == 0 else _sc_calls[p](C, ids, out_t)
    return out_t.T


# speedup vs baseline: 1.2516x; 1.0022x over previous
import functools

import jax
import jax.numpy as jnp
from jax import lax
from jax.experimental import pallas as pl
from jax.experimental.pallas import tpu as pltpu
from jax.experimental.pallas import tpu_sc as plsc
from jax._src.pallas import mpmd as _pl_mpmd

N = 131072
D = 32
K = 512

P = 2
NP = N // P


R = 1024
NBP = NP // R


NC = 2
NS = 16
NW = NC * NS
RPW = NP // NW
RPC = 512
NCH = RPW // RPC
GPC = RPC // 16


def _dist_argmin_kernel(xt_ref, c_ref, ids_ref):
    xt = xt_ref[...]
    c = c_ref[...]
    cnorm = jnp.sum(c * c, axis=0, keepdims=True)
    xc2 = lax.dot_general(
        xt, -2.0 * c, (((0,), (0,)), ((), ())),
        preferred_element_type=jnp.float32,
    )
    dist = xc2 + cnorm
    ids_ref[...] = jnp.argmin(dist, axis=1).astype(jnp.int32)


def _make_dist_argmin(phase):
    return pl.pallas_call(
        _dist_argmin_kernel,
        grid=(NBP,),
        in_specs=[
            pl.BlockSpec((D, R), lambda i, o=phase * NBP: (0, i + o)),
            pl.BlockSpec((D, K), lambda i: (0, 0)),
        ],
        out_specs=pl.BlockSpec((R,), lambda i: (i,)),
        out_shape=jax.ShapeDtypeStruct((NP,), jnp.int32),
        compiler_params=pltpu.CompilerParams(fuse_transposed_lhs_in_matmul=True),
    )


_sc_mesh = plsc.VectorSubcoreMesh(core_axis_name="c", subcore_axis_name="s")


def _make_gather(phase):
    col_base = phase * NP
    aliased = phase > 0

    def _gather_codewords(table_hbm, idx_hbm, *rest):
        if aliased:
            _, out_hbm, table_v, idx_v, out_c = rest
        else:
            out_hbm, table_v, idx_v, out_c = rest
        wid = lax.axis_index("s") * NC + lax.axis_index("c")
        pltpu.sync_copy(table_hbm, table_v)
        pltpu.sync_copy(idx_hbm.at[pl.ds(wid * RPW, RPW)], idx_v)

        def chunk_body(k, carry):
            @plsc.parallel_loop(0, GPC, unroll=2)
            def group_body(g):
                ids16 = idx_v[pl.ds(k * RPC + g * 16, 16)]
                for col in range(D):
                    col_vec = jnp.full((16,), col, jnp.int32)
                    v = plsc.load_gather(table_v, [col_vec, ids16])
                    out_c[col, pl.ds(g * 16, 16)] = v

            col0 = col_base + (wid * NCH + k) * RPC
            pltpu.sync_copy(out_c, out_hbm.at[:, pl.ds(col0, RPC)])
            return carry

        lax.fori_loop(0, NCH, chunk_body, 0)

    return _pl_mpmd._mpmd_map(
        [(_sc_mesh, _gather_codewords)],
        jax.ShapeDtypeStruct((D, N), jnp.float32),
        input_output_aliases={2: 0} if aliased else {},
        scratch_types=[
            pltpu.VMEM((D, K), jnp.float32),
            pltpu.VMEM((RPW,), jnp.int32),
            pltpu.VMEM((D, RPC), jnp.float32),
        ],
        compiler_params=pltpu.CompilerParams(needs_layout_passes=False),
    )


_tc_calls = [_make_dist_argmin(p) for p in range(P)]
_sc_calls = [_make_gather(p) for p in range(P)]


def kernel(x, C):
    xt = x.T
    out_t = None
    for p in range(P):
        ids = _tc_calls[p](xt, C)
        out_t = _sc_calls[p](C, ids) if p == 0 else _sc_calls[p](C, ids, out_t)
    return out_t.T


# speedup vs baseline: 2.7745x; 2.2168x over previous
import functools

import jax
import jax.numpy as jnp
from jax import lax
from jax.experimental import pallas as pl
from jax.experimental.pallas import tpu as pltpu
from jax.experimental.pallas import tpu_sc as plsc
from jax._src.pallas import mpmd as _pl_mpmd

N = 131072
D = 32
K = 512

P = 2
NP = N // P


R = 1024
NBP = NP // R


NC = 2
NS = 16
NW = NC * NS
RPW = NP // NW
RPC = 512
NCH = RPW // RPC
GPC = RPC // 16


def _dist_argmin_kernel(xt_ref, c_ref, ids_ref):
    xt = xt_ref[...]
    c = c_ref[...]
    ct = c.T
    cnorm_col = jnp.sum(ct * ct, axis=1, keepdims=True)
    xc2t = jnp.dot(-2.0 * ct, xt, preferred_element_type=jnp.float32)
    dist_t = xc2t + cnorm_col
    ids_ref[...] = jnp.argmin(dist_t, axis=0).astype(jnp.int32)


def _make_dist_argmin(phase):
    return pl.pallas_call(
        _dist_argmin_kernel,
        grid=(NBP,),
        in_specs=[
            pl.BlockSpec((D, R), lambda i, o=phase * NBP: (0, i + o)),
            pl.BlockSpec((D, K), lambda i: (0, 0)),
        ],
        out_specs=pl.BlockSpec((R,), lambda i: (i,)),
        out_shape=jax.ShapeDtypeStruct((NP,), jnp.int32),
        compiler_params=pltpu.CompilerParams(fuse_transposed_lhs_in_matmul=True),
    )


_sc_mesh = plsc.VectorSubcoreMesh(core_axis_name="c", subcore_axis_name="s")


def _make_gather(phase):
    col_base = phase * NP
    aliased = phase > 0

    def _gather_codewords(table_hbm, idx_hbm, *rest):
        if aliased:
            _, out_hbm, table_v, idx_v, out_c = rest
        else:
            out_hbm, table_v, idx_v, out_c = rest
        wid = lax.axis_index("s") * NC + lax.axis_index("c")
        pltpu.sync_copy(table_hbm, table_v)
        pltpu.sync_copy(idx_hbm.at[pl.ds(wid * RPW, RPW)], idx_v)

        def chunk_body(k, carry):
            @plsc.parallel_loop(0, GPC, unroll=2)
            def group_body(g):
                ids16 = idx_v[pl.ds(k * RPC + g * 16, 16)]
                for col in range(D):
                    col_vec = jnp.full((16,), col, jnp.int32)
                    v = plsc.load_gather(table_v, [col_vec, ids16])
                    out_c[col, pl.ds(g * 16, 16)] = v

            col0 = col_base + (wid * NCH + k) * RPC
            pltpu.sync_copy(out_c, out_hbm.at[:, pl.ds(col0, RPC)])
            return carry

        lax.fori_loop(0, NCH, chunk_body, 0)

    return _pl_mpmd._mpmd_map(
        [(_sc_mesh, _gather_codewords)],
        jax.ShapeDtypeStruct((D, N), jnp.float32),
        input_output_aliases={2: 0} if aliased else {},
        scratch_types=[
            pltpu.VMEM((D, K), jnp.float32),
            pltpu.VMEM((RPW,), jnp.int32),
            pltpu.VMEM((D, RPC), jnp.float32),
        ],
        compiler_params=pltpu.CompilerParams(needs_layout_passes=False),
    )


_tc_calls = [_make_dist_argmin(p) for p in range(P)]
_sc_calls = [_make_gather(p) for p in range(P)]


def kernel(x, C):
    xt = x.T
    out_t = None
    for p in range(P):
        ids = _tc_calls[p](xt, C)
        out_t = _sc_calls[p](C, ids) if p == 0 else _sc_calls[p](C, ids, out_t)
    return out_t.T


# speedup vs baseline: 2.8362x; 1.0223x over previous
import functools

import jax
import jax.numpy as jnp
from jax import lax
from jax.experimental import pallas as pl
from jax.experimental.pallas import tpu as pltpu
from jax.experimental.pallas import tpu_sc as plsc
from jax._src.pallas import mpmd as _pl_mpmd

N = 131072
D = 32
K = 512

PHASES = (81920, 32768, 16384)
assert sum(PHASES) == N


R = 1024


NC = 2
NS = 16
NW = NC * NS
RPC = 512
GPC = RPC // 16


def _dist_argmin_kernel(xt_ref, c_ref, ids_ref):
    xt = xt_ref[...]
    c = c_ref[...]
    ct = c.T
    cnorm_col = jnp.sum(ct * ct, axis=1, keepdims=True)
    xc2t = jnp.dot(-2.0 * ct, xt, preferred_element_type=jnp.float32)
    dist_t = xc2t + cnorm_col
    ids_ref[...] = jnp.argmin(dist_t, axis=0).astype(jnp.int32)


def _make_dist_argmin(row_off, rows):
    return pl.pallas_call(
        _dist_argmin_kernel,
        grid=(rows // R,),
        in_specs=[
            pl.BlockSpec((D, R), lambda i, o=row_off // R: (0, i + o)),
            pl.BlockSpec((D, K), lambda i: (0, 0)),
        ],
        out_specs=pl.BlockSpec((R,), lambda i: (i,)),
        out_shape=jax.ShapeDtypeStruct((rows,), jnp.int32),
    )


_sc_mesh = plsc.VectorSubcoreMesh(core_axis_name="c", subcore_axis_name="s")


def _make_gather(row_off, rows, aliased):
    rpw = rows // NW
    nch = rpw // RPC

    def _gather_codewords(table_hbm, idx_hbm, *rest):
        if aliased:
            _, out_hbm, table_v, idx_v, out_c = rest
        else:
            out_hbm, table_v, idx_v, out_c = rest
        wid = lax.axis_index("s") * NC + lax.axis_index("c")
        pltpu.sync_copy(table_hbm, table_v)
        pltpu.sync_copy(idx_hbm.at[pl.ds(wid * rpw, rpw)], idx_v)

        def chunk_body(k, carry):
            @plsc.parallel_loop(0, GPC, unroll=2)
            def group_body(g):
                ids16 = idx_v[pl.ds(k * RPC + g * 16, 16)]
                for col in range(D):
                    col_vec = jnp.full((16,), col, jnp.int32)
                    v = plsc.load_gather(table_v, [col_vec, ids16])
                    out_c[col, pl.ds(g * 16, 16)] = v

            col0 = row_off + (wid * nch + k) * RPC
            pltpu.sync_copy(out_c, out_hbm.at[:, pl.ds(col0, RPC)])
            return carry

        lax.fori_loop(0, nch, chunk_body, 0)

    return _pl_mpmd._mpmd_map(
        [(_sc_mesh, _gather_codewords)],
        jax.ShapeDtypeStruct((D, N), jnp.float32),
        input_output_aliases={2: 0} if aliased else {},
        scratch_types=[
            pltpu.VMEM((D, K), jnp.float32),
            pltpu.VMEM((rpw,), jnp.int32),
            pltpu.VMEM((D, RPC), jnp.float32),
        ],
        compiler_params=pltpu.CompilerParams(needs_layout_passes=False),
    )


_offsets = [sum(PHASES[:p]) for p in range(len(PHASES))]
_tc_calls = [_make_dist_argmin(o, n) for o, n in zip(_offsets, PHASES)]
_sc_calls = [_make_gather(o, n, p > 0)
             for p, (o, n) in enumerate(zip(_offsets, PHASES))]


def kernel(x, C):
    xt = x.T
    out_t = None
    for p in range(len(PHASES)):
        ids = _tc_calls[p](xt, C)
        out_t = _sc_calls[p](C, ids) if p == 0 else _sc_calls[p](C, ids, out_t)
    return out_t.T
